# Initial kernel scaffold; baseline (speedup 1.0000x reference)
import jax
import jax.numpy as jnp
from jax.experimental import pallas as pl


def kernel(gene_expr, coords, W_g1, b_g1, W_c1, b_c1, W_msg, b_msg, W_self, b_self, W_z, b_z):
    raise NotImplementedError("write your pallas kernel here")


# speedup vs baseline: 6.4824x; 6.4824x over previous
import functools

import jax
import jax.numpy as jnp
from jax import lax
from jax.experimental import pallas as pl

N = 4096
G = 512
H = 512
L = 128
KSEL = 9
KPAD = 16
RBF_SIGMA = 1.0
EPS = 1e-08
BN = 512
NB = N // BN


def _enc_body(gene_ref, wg_ref, bg_ref, cb_ref, wc_ref, bc_ref, h_ref):
    x = gene_ref[...]
    hg = jnp.maximum(jnp.dot(x, wg_ref[...], preferred_element_type=jnp.float32) + bg_ref[...], 0.0)
    cb = cb_ref[...]
    wc = wc_ref[...]
    hc = cb[:, 0:1] * wc[0:1, :] + cb[:, 1:2] * wc[1:2, :] + bc_ref[...]
    h_ref[...] = hg + jnp.maximum(hc, 0.0)


def _knn_body(cb_ref, ct_ref, idx_ref, w_ref, idxT_ref, wT_ref):
    b = pl.program_id(0)
    cb = cb_ref[...]
    ct = ct_ref[...]
    sqb = cb[:, 0:1] * cb[:, 0:1] + cb[:, 1:2] * cb[:, 1:2]
    sqa = ct[0:1, :] * ct[0:1, :] + ct[1:2, :] * ct[1:2, :]
    cross = jnp.dot(cb, ct, preferred_element_type=jnp.float32)
    d2 = (sqb + sqa) - 2.0 * cross
    s = jnp.sqrt(jnp.maximum(d2, 1e-12))
    cols = lax.broadcasted_iota(jnp.int32, (BN, N), 1)
    rows = lax.broadcasted_iota(jnp.int32, (BN, 1), 0) + b * BN
    denom = jnp.float32(2.0 * RBF_SIGMA ** 2 + 1e-12)
    idx_cols = []
    w_cols = []
    for _ in range(KSEL):
        m = jnp.min(s, axis=1, keepdims=True)
        am = jnp.min(jnp.where(s == m, cols, N), axis=1, keepdims=True)
        wt = jnp.exp(-(m * m) / denom)
        wt = jnp.where(am == rows, 0.0, wt)
        idx_cols.append(am)
        w_cols.append(wt)
        s = jnp.where(cols == am, jnp.float32(jnp.inf), s)
    zero = jnp.zeros((BN, 1), jnp.float32)
    for _ in range(KSEL, KPAD):
        idx_cols.append(rows)
        w_cols.append(zero)
    idx_blk = jnp.concatenate(idx_cols, axis=1)
    w_blk = jnp.concatenate(w_cols, axis=1)
    idx_ref[...] = idx_blk
    w_ref[...] = w_blk
    idxT_ref[...] = idx_blk.T
    wT_ref[...] = w_blk.T


def _deg_body(w_ref, idxT_ref, wT_ref, dinvc_ref, dinvr_ref):
    b = pl.program_id(0)
    rows = lax.broadcasted_iota(jnp.int32, (BN, 1), 0) + b * BN
    idxT = idxT_ref[...]
    wT = wT_ref[...]
    rev = jnp.sum(w_ref[...], axis=1, keepdims=True)
    for t in range(KSEL):
        contrib = jnp.where(idxT[t:t + 1, :] == rows, wT[t:t + 1, :], 0.0)
        rev = rev + jnp.sum(contrib, axis=1, keepdims=True)
    deg = 0.5 * rev
    dinv = lax.rsqrt(deg + EPS)
    dinvc_ref[...] = dinv
    dinvr_ref[...] = dinv.T


def _agg_body(idx_ref, w_ref, idxT_ref, wT_ref, dinvr_ref, dinvc_ref, h_ref,
              hb_ref, wmsg_ref, bmsg_ref, wself_ref, bself_ref, wz_ref,
              bz_ref, z_ref):
    b = pl.program_id(0)
    cols = lax.broadcasted_iota(jnp.int32, (BN, N), 1)
    rows = lax.broadcasted_iota(jnp.int32, (BN, 1), 0) + b * BN
    idxb = idx_ref[...]
    wb = w_ref[...]
    idxT = idxT_ref[...]
    wT = wT_ref[...]
    A = jnp.zeros((BN, N), jnp.float32)
    for t in range(KSEL):
        A = A + jnp.where(idxb[:, t:t + 1] == cols, wb[:, t:t + 1], 0.0)
        A = A + jnp.where(idxT[t:t + 1, :] == rows, wT[t:t + 1, :], 0.0)
    A = A * (0.5 * dinvc_ref[...])
    A = A * dinvr_ref[...]
    agg = jnp.dot(A, h_ref[...], preferred_element_type=jnp.float32)
    msg = jnp.maximum(
        jnp.dot(agg, wmsg_ref[...], preferred_element_type=jnp.float32)
        + bmsg_ref[...], 0.0)
    selff = jnp.maximum(
        jnp.dot(hb_ref[...], wself_ref[...], preferred_element_type=jnp.float32)
        + bself_ref[...], 0.0)
    hf = jnp.maximum(msg + selff, 0.0)
    z_ref[...] = jnp.dot(hf, wz_ref[...], preferred_element_type=jnp.float32) + bz_ref[...]


def _full(shape):
    nd = len(shape)
    return pl.BlockSpec(shape, lambda b: (0,) * nd)


def _rowblk(shape):
    nd = len(shape)
    return pl.BlockSpec(shape, lambda b: (b,) + (0,) * (nd - 1))


@jax.jit
def kernel(gene_expr, coords, W_g1, b_g1, W_c1, b_c1, W_msg, b_msg, W_self,
           b_self, W_z, b_z):
    f32 = jnp.float32
    coordsT = coords.T
    wgT = W_g1.T
    wcT = W_c1.T
    wmsgT = W_msg.T
    wselfT = W_self.T
    wzT = W_z.T
    bg = b_g1.reshape(1, H)
    bc = b_c1.reshape(1, H)
    bmsg = b_msg.reshape(1, H)
    bself = b_self.reshape(1, H)
    bz = b_z.reshape(1, L)

    h = pl.pallas_call(
        _enc_body,
        grid=(NB,),
        in_specs=[_rowblk((BN, G)), _full((G, H)), _full((1, H)),
                  _rowblk((BN, 2)), _full((2, H)), _full((1, H))],
        out_specs=_rowblk((BN, H)),
        out_shape=jax.ShapeDtypeStruct((N, H), f32),
    )(gene_expr, wgT, bg, coords, wcT, bc)

    idx, w, idxT, wT = pl.pallas_call(
        _knn_body,
        grid=(NB,),
        in_specs=[_rowblk((BN, 2)), _full((2, N))],
        out_specs=[_rowblk((BN, KPAD)), _rowblk((BN, KPAD)),
                   pl.BlockSpec((KPAD, BN), lambda b: (0, b)),
                   pl.BlockSpec((KPAD, BN), lambda b: (0, b))],
        out_shape=[jax.ShapeDtypeStruct((N, KPAD), jnp.int32),
                   jax.ShapeDtypeStruct((N, KPAD), f32),
                   jax.ShapeDtypeStruct((KPAD, N), jnp.int32),
                   jax.ShapeDtypeStruct((KPAD, N), f32)],
    )(coords, coordsT)

    dinv_col, dinv_row = pl.pallas_call(
        _deg_body,
        grid=(NB,),
        in_specs=[_rowblk((BN, KPAD)), _full((KPAD, N)), _full((KPAD, N))],
        out_specs=[_rowblk((BN, 1)), pl.BlockSpec((1, BN), lambda b: (0, b))],
        out_shape=[jax.ShapeDtypeStruct((N, 1), f32),
                   jax.ShapeDtypeStruct((1, N), f32)],
    )(w, idxT, wT)

    z = pl.pallas_call(
        _agg_body,
        grid=(NB,),
        in_specs=[_rowblk((BN, KPAD)), _rowblk((BN, KPAD)), _full((KPAD, N)),
                  _full((KPAD, N)), _full((1, N)), _rowblk((BN, 1)),
                  _full((N, H)), _rowblk((BN, H)), _full((H, H)),
                  _full((1, H)), _full((H, H)), _full((1, H)), _full((H, L)),
                  _full((1, L))],
        out_specs=_rowblk((BN, L)),
        out_shape=jax.ShapeDtypeStruct((N, L), f32),
    )(idx, w, idxT, wT, dinv_row, dinv_col, h, h, wmsgT, bmsg, wselfT, bself,
      wzT, bz)
    return z


# speedup vs baseline: 6.7935x; 1.0480x over previous
import functools

import jax
import jax.numpy as jnp
from jax import lax
from jax.experimental import pallas as pl
from jax.experimental.pallas import tpu as pltpu
from jax.experimental.pallas import tpu_sc as plsc

N = 4096
G = 512
H = 512
L = 128
KSEL = 9
KPAD = 16
RBF_SIGMA = 1.0
EPS = 1e-08
BN = 512
NB = N // BN


def _enc_body(gene_ref, wg_ref, bg_ref, cb_ref, wc_ref, bc_ref, h_ref):
    x = gene_ref[...]
    hg = jnp.maximum(jnp.dot(x, wg_ref[...], preferred_element_type=jnp.float32) + bg_ref[...], 0.0)
    cb = cb_ref[...]
    wc = wc_ref[...]
    hc = cb[:, 0:1] * wc[0:1, :] + cb[:, 1:2] * wc[1:2, :] + bc_ref[...]
    h_ref[...] = hg + jnp.maximum(hc, 0.0)


def _knn_body(cb_ref, ct_ref, idx_ref, w_ref, idxT_ref, wT_ref, rs_ref):
    b = pl.program_id(0)
    cb = cb_ref[...]
    ct = ct_ref[...]
    sqb = cb[:, 0:1] * cb[:, 0:1] + cb[:, 1:2] * cb[:, 1:2]
    sqa = ct[0:1, :] * ct[0:1, :] + ct[1:2, :] * ct[1:2, :]
    cross = jnp.dot(cb, ct, preferred_element_type=jnp.float32)
    d2 = (sqb + sqa) - 2.0 * cross
    s = jnp.sqrt(jnp.maximum(d2, 1e-12))
    cols = lax.broadcasted_iota(jnp.int32, (BN, N), 1)
    rows = lax.broadcasted_iota(jnp.int32, (BN, 1), 0) + b * BN
    denom = jnp.float32(2.0 * RBF_SIGMA ** 2 + 1e-12)
    big = jnp.float32(1e30)
    idx_cols = []
    w_cols = []
    for _ in range(KSEL):
        m = jnp.min(s, axis=1, keepdims=True)
        am = jnp.min(jnp.where(s == m, cols, N), axis=1, keepdims=True)
        s = jnp.where(cols == am, big, s)
        wt = jnp.exp(-(m * m) / denom)
        wt = jnp.where(am == rows, 0.0, wt)
        idx_cols.append(am)
        w_cols.append(wt)
    zero = jnp.zeros((BN, 1), jnp.float32)
    rowsum = w_cols[0]
    for t in range(1, KSEL):
        rowsum = rowsum + w_cols[t]
    for _ in range(KSEL, KPAD):
        idx_cols.append(rows)
        w_cols.append(zero)
    idx_blk = jnp.concatenate(idx_cols, axis=1)
    w_blk = jnp.concatenate(w_cols, axis=1)
    idx_ref[...] = idx_blk
    w_ref[...] = w_blk
    idxT_ref[...] = idx_blk.T
    wT_ref[...] = w_blk.T
    rs_ref[...] = rowsum


def _sc_deg_body(idx_hbm, w_hbm, out_hbm, idx_v, w_v, accf_v, acc_v):
    cid = lax.axis_index("c")
    sid = lax.axis_index("s")
    wid = cid * 16 + sid
    base = wid * (N // 32)
    pltpu.sync_copy(idx_hbm.at[pl.ds(base, N // 32)], idx_v)
    pltpu.sync_copy(w_hbm.at[pl.ds(base, N // 32)], w_v)

    def _z(i, c):
        accf_v[pl.ds(i * 16, 16)] = jnp.zeros((16,), jnp.float32)
        return c

    lax.fori_loop(0, 256, _z, 0)

    def _e(e, c):
        plsc.addupdate_scatter(accf_v, [idx_v[e, :]], w_v[e, :])
        return c

    lax.fori_loop(0, N // 32, _e, 0)

    def _rp(i, c):
        acc_v[i, :] = accf_v[pl.ds(i * 16, 16)]
        return c

    lax.fori_loop(0, 256, _rp, 0)

    pltpu.sync_copy(acc_v, out_hbm.at[wid])


def _sc_comb_body(parts_hbm, out_hbm, buf_v, acc2_v):
    cid = lax.axis_index("c")
    sid = lax.axis_index("s")
    wid = cid * 16 + sid
    base = wid * 8

    def _z(i, c):
        acc2_v[i, :] = jnp.zeros((16,), jnp.float32)
        return c

    lax.fori_loop(0, 8, _z, 0)
    for t in range(32):
        pltpu.sync_copy(parts_hbm.at[t, pl.ds(base, 8)], buf_v)

        def _add(r, c2):
            acc2_v[r, :] = acc2_v[r, :] + buf_v[r, :]
            return c2

        lax.fori_loop(0, 8, _add, 0)
    pltpu.sync_copy(acc2_v, out_hbm.at[pl.ds(base, 8)])


def _sc_deg(idx, w):
    mesh = plsc.VectorSubcoreMesh(core_axis_name="c", subcore_axis_name="s")
    parts = pl.kernel(
        _sc_deg_body,
        out_type=jax.ShapeDtypeStruct((32, 256, 16), jnp.float32),
        mesh=mesh,
        scratch_types=[
            pltpu.VMEM((N // 32, KPAD), jnp.int32),
            pltpu.VMEM((N // 32, KPAD), jnp.float32),
            pltpu.VMEM((N,), jnp.float32),
            pltpu.VMEM((256, 16), jnp.float32),
        ],
        compiler_params=pltpu.CompilerParams(needs_layout_passes=False),
    )(idx, w)
    colsum = pl.kernel(
        _sc_comb_body,
        out_type=jax.ShapeDtypeStruct((256, 16), jnp.float32),
        mesh=mesh,
        scratch_types=[
            pltpu.VMEM((8, 16), jnp.float32),
            pltpu.VMEM((8, 16), jnp.float32),
        ],
        compiler_params=pltpu.CompilerParams(needs_layout_passes=False),
    )(parts)
    return colsum


def _agg_body(idx_ref, w_ref, idxT_ref, wT_ref, dinvr_ref, dinvc_ref, h_ref,
              hb_ref, wmsg_ref, bmsg_ref, wself_ref, bself_ref, wz_ref,
              bz_ref, z_ref):
    b = pl.program_id(0)
    cols = lax.broadcasted_iota(jnp.int32, (BN, N), 1)
    rows = lax.broadcasted_iota(jnp.int32, (BN, 1), 0) + b * BN
    idxb = idx_ref[...]
    wb = w_ref[...]
    idxT = idxT_ref[...]
    wT = wT_ref[...]
    A = jnp.zeros((BN, N), jnp.float32)
    for t in range(KSEL):
        A = A + jnp.where(idxb[:, t:t + 1] == cols, wb[:, t:t + 1], 0.0)
        A = A + jnp.where(idxT[t:t + 1, :] == rows, wT[t:t + 1, :], 0.0)
    A = A * (0.5 * dinvc_ref[...])
    A = A * dinvr_ref[...]
    agg = jnp.dot(A, h_ref[...], preferred_element_type=jnp.float32)
    msg = jnp.maximum(
        jnp.dot(agg, wmsg_ref[...], preferred_element_type=jnp.float32)
        + bmsg_ref[...], 0.0)
    selff = jnp.maximum(
        jnp.dot(hb_ref[...], wself_ref[...], preferred_element_type=jnp.float32)
        + bself_ref[...], 0.0)
    hf = jnp.maximum(msg + selff, 0.0)
    z_ref[...] = jnp.dot(hf, wz_ref[...], preferred_element_type=jnp.float32) + bz_ref[...]


def _full(shape):
    nd = len(shape)
    return pl.BlockSpec(shape, lambda b: (0,) * nd)


def _rowblk(shape):
    nd = len(shape)
    return pl.BlockSpec(shape, lambda b: (b,) + (0,) * (nd - 1))


@jax.jit
def kernel(gene_expr, coords, W_g1, b_g1, W_c1, b_c1, W_msg, b_msg, W_self,
           b_self, W_z, b_z):
    f32 = jnp.float32
    coordsT = coords.T
    wgT = W_g1.T
    wcT = W_c1.T
    wmsgT = W_msg.T
    wselfT = W_self.T
    wzT = W_z.T
    bg = b_g1.reshape(1, H)
    bc = b_c1.reshape(1, H)
    bmsg = b_msg.reshape(1, H)
    bself = b_self.reshape(1, H)
    bz = b_z.reshape(1, L)

    h = pl.pallas_call(
        _enc_body,
        grid=(NB,),
        in_specs=[_rowblk((BN, G)), _full((G, H)), _full((1, H)),
                  _rowblk((BN, 2)), _full((2, H)), _full((1, H))],
        out_specs=_rowblk((BN, H)),
        out_shape=jax.ShapeDtypeStruct((N, H), f32),
    )(gene_expr, wgT, bg, coords, wcT, bc)

    idx, w, idxT, wT, rowsum = pl.pallas_call(
        _knn_body,
        grid=(NB,),
        in_specs=[_rowblk((BN, 2)), _full((2, N))],
        out_specs=[_rowblk((BN, KPAD)), _rowblk((BN, KPAD)),
                   pl.BlockSpec((KPAD, BN), lambda b: (0, b)),
                   pl.BlockSpec((KPAD, BN), lambda b: (0, b)),
                   _rowblk((BN, 1))],
        out_shape=[jax.ShapeDtypeStruct((N, KPAD), jnp.int32),
                   jax.ShapeDtypeStruct((N, KPAD), f32),
                   jax.ShapeDtypeStruct((KPAD, N), jnp.int32),
                   jax.ShapeDtypeStruct((KPAD, N), f32),
                   jax.ShapeDtypeStruct((N, 1), f32)],
    )(coords, coordsT)

    colsum = _sc_deg(idx, w)
    deg = 0.5 * (rowsum.reshape(N) + colsum.reshape(N))
    dinv = jnp.power(deg + EPS, -0.5)
    dinv_col = dinv.reshape(N, 1)
    dinv_row = dinv.reshape(1, N)

    z = pl.pallas_call(
        _agg_body,
        grid=(NB,),
        in_specs=[_rowblk((BN, KPAD)), _rowblk((BN, KPAD)), _full((KPAD, N)),
                  _full((KPAD, N)), _full((1, N)), _rowblk((BN, 1)),
                  _full((N, H)), _rowblk((BN, H)), _full((H, H)),
                  _full((1, H)), _full((H, H)), _full((1, H)), _full((H, L)),
                  _full((1, L))],
        out_specs=_rowblk((BN, L)),
        out_shape=jax.ShapeDtypeStruct((N, L), f32),
    )(idx, w, idxT, wT, dinv_row, dinv_col, h, h, wmsgT, bmsg, wselfT, bself,
      wzT, bz)
    return z


# speedup vs baseline: 6.7967x; 1.0005x over previous
import functools

import jax
import jax.numpy as jnp
from jax import lax
from jax.experimental import pallas as pl
from jax.experimental.pallas import tpu as pltpu
from jax.experimental.pallas import tpu_sc as plsc

N = 4096
G = 512
H = 512
L = 128
KSEL = 9
KPAD = 16
RBF_SIGMA = 1.0
EPS = 1e-08
BN = 512
NB = N // BN


def _enc_body(gene_ref, wg_ref, bg_ref, cb_ref, wc_ref, bc_ref, h_ref):
    x = gene_ref[...]
    hg = jnp.maximum(jnp.dot(x, wg_ref[...], preferred_element_type=jnp.float32) + bg_ref[...], 0.0)
    cb = cb_ref[...]
    wc = wc_ref[...]
    hc = cb[:, 0:1] * wc[0:1, :] + cb[:, 1:2] * wc[1:2, :] + bc_ref[...]
    h_ref[...] = hg + jnp.maximum(hc, 0.0)


def _knn_body(cb_ref, ct_ref, idx_ref, w_ref, idxT_ref, wT_ref, rs_ref):
    b = pl.program_id(0)
    cb = cb_ref[...]
    ct = ct_ref[...]
    sqb = cb[:, 0:1] * cb[:, 0:1] + cb[:, 1:2] * cb[:, 1:2]
    sqa = ct[0:1, :] * ct[0:1, :] + ct[1:2, :] * ct[1:2, :]
    cross = jnp.dot(cb, ct, preferred_element_type=jnp.float32)
    d2 = (sqb + sqa) - 2.0 * cross
    s = jnp.sqrt(jnp.maximum(d2, 1e-12))
    cols = lax.broadcasted_iota(jnp.int32, (BN, N), 1)
    rows = lax.broadcasted_iota(jnp.int32, (BN, 1), 0) + b * BN
    denom = jnp.float32(2.0 * RBF_SIGMA ** 2 + 1e-12)
    big = jnp.float32(1e30)
    idx_cols = []
    w_cols = []
    for t in range(KSEL):
        m = jnp.min(s, axis=1, keepdims=True)
        am = jnp.min(jnp.where(s == m, cols, N), axis=1, keepdims=True)
        if t + 1 < KSEL:
            s = jnp.where(cols == am, big, s)
        wt = jnp.exp(-(m * m) / denom)
        wt = jnp.where(am == rows, 0.0, wt)
        idx_cols.append(am)
        w_cols.append(wt)
    zero = jnp.zeros((BN, 1), jnp.float32)
    rowsum = w_cols[0]
    for t in range(1, KSEL):
        rowsum = rowsum + w_cols[t]
    for _ in range(KSEL, KPAD):
        idx_cols.append(rows)
        w_cols.append(zero)
    idx_blk = jnp.concatenate(idx_cols, axis=1)
    w_blk = jnp.concatenate(w_cols, axis=1)
    idx_ref[...] = idx_blk
    w_ref[...] = w_blk
    idxT_ref[...] = idx_blk.T
    wT_ref[...] = w_blk.T
    rs_ref[...] = rowsum


def _sc_deg_body(idx_hbm, w_hbm, out_hbm, idx_v, w_v, accf_v, acc_v):
    cid = lax.axis_index("c")
    sid = lax.axis_index("s")
    wid = cid * 16 + sid
    base = wid * (N // 32)
    pltpu.sync_copy(idx_hbm.at[pl.ds(base, N // 32)], idx_v)
    pltpu.sync_copy(w_hbm.at[pl.ds(base, N // 32)], w_v)

    def _z(i, c):
        accf_v[pl.ds(i * 16, 16)] = jnp.zeros((16,), jnp.float32)
        return c

    lax.fori_loop(0, 256, _z, 0)

    def _e(e, c):
        plsc.addupdate_scatter(accf_v, [idx_v[e, :]], w_v[e, :])
        return c

    lax.fori_loop(0, N // 32, _e, 0)

    def _rp(i, c):
        acc_v[i, :] = accf_v[pl.ds(i * 16, 16)]
        return c

    lax.fori_loop(0, 256, _rp, 0)

    pltpu.sync_copy(acc_v, out_hbm.at[wid])


def _sc_comb_body(parts_hbm, out_hbm, buf_v, acc2_v):
    cid = lax.axis_index("c")
    sid = lax.axis_index("s")
    wid = cid * 16 + sid
    base = wid * 8

    def _z(i, c):
        acc2_v[i, :] = jnp.zeros((16,), jnp.float32)
        return c

    lax.fori_loop(0, 8, _z, 0)
    for t in range(32):
        pltpu.sync_copy(parts_hbm.at[t, pl.ds(base, 8)], buf_v)

        def _add(r, c2):
            acc2_v[r, :] = acc2_v[r, :] + buf_v[r, :]
            return c2

        lax.fori_loop(0, 8, _add, 0)
    pltpu.sync_copy(acc2_v, out_hbm.at[pl.ds(base, 8)])


def _sc_deg(idx, w):
    mesh = plsc.VectorSubcoreMesh(core_axis_name="c", subcore_axis_name="s")
    parts = pl.kernel(
        _sc_deg_body,
        out_type=jax.ShapeDtypeStruct((32, 256, 16), jnp.float32),
        mesh=mesh,
        scratch_types=[
            pltpu.VMEM((N // 32, KPAD), jnp.int32),
            pltpu.VMEM((N // 32, KPAD), jnp.float32),
            pltpu.VMEM((N,), jnp.float32),
            pltpu.VMEM((256, 16), jnp.float32),
        ],
        compiler_params=pltpu.CompilerParams(needs_layout_passes=False),
    )(idx, w)
    colsum = pl.kernel(
        _sc_comb_body,
        out_type=jax.ShapeDtypeStruct((256, 16), jnp.float32),
        mesh=mesh,
        scratch_types=[
            pltpu.VMEM((8, 16), jnp.float32),
            pltpu.VMEM((8, 16), jnp.float32),
        ],
        compiler_params=pltpu.CompilerParams(needs_layout_passes=False),
    )(parts)
    return colsum


def _agg_body(idx_ref, w_ref, idxT_ref, wT_ref, dinvr_ref, dinvc_ref, h_ref,
              hb_ref, wmsg_ref, bmsg_ref, wself_ref, bself_ref, wz_ref,
              bz_ref, z_ref):
    b = pl.program_id(0)
    cols = lax.broadcasted_iota(jnp.int32, (BN, N), 1)
    rows = lax.broadcasted_iota(jnp.int32, (BN, 1), 0) + b * BN
    idxb = idx_ref[...]
    wb = w_ref[...]
    idxT = idxT_ref[...]
    wT = wT_ref[...]
    A = jnp.zeros((BN, N), jnp.float32)
    for t in range(KSEL):
        A = A + jnp.where(idxb[:, t:t + 1] == cols, wb[:, t:t + 1], 0.0)
        A = A + jnp.where(idxT[t:t + 1, :] == rows, wT[t:t + 1, :], 0.0)
    A = A * (0.5 * dinvc_ref[...])
    A = A * dinvr_ref[...]
    agg = jnp.dot(A, h_ref[...], preferred_element_type=jnp.float32)
    msg = jnp.maximum(
        jnp.dot(agg, wmsg_ref[...], preferred_element_type=jnp.float32)
        + bmsg_ref[...], 0.0)
    selff = jnp.maximum(
        jnp.dot(hb_ref[...], wself_ref[...], preferred_element_type=jnp.float32)
        + bself_ref[...], 0.0)
    hf = jnp.maximum(msg + selff, 0.0)
    z_ref[...] = jnp.dot(hf, wz_ref[...], preferred_element_type=jnp.float32) + bz_ref[...]


def _full(shape):
    nd = len(shape)
    return pl.BlockSpec(shape, lambda b: (0,) * nd)


def _rowblk(shape):
    nd = len(shape)
    return pl.BlockSpec(shape, lambda b: (b,) + (0,) * (nd - 1))


@jax.jit
def kernel(gene_expr, coords, W_g1, b_g1, W_c1, b_c1, W_msg, b_msg, W_self,
           b_self, W_z, b_z):
    f32 = jnp.float32
    coordsT = coords.T
    wgT = W_g1.T
    wcT = W_c1.T
    wmsgT = W_msg.T
    wselfT = W_self.T
    wzT = W_z.T
    bg = b_g1.reshape(1, H)
    bc = b_c1.reshape(1, H)
    bmsg = b_msg.reshape(1, H)
    bself = b_self.reshape(1, H)
    bz = b_z.reshape(1, L)

    h = pl.pallas_call(
        _enc_body,
        grid=(NB,),
        in_specs=[_rowblk((BN, G)), _full((G, H)), _full((1, H)),
                  _rowblk((BN, 2)), _full((2, H)), _full((1, H))],
        out_specs=_rowblk((BN, H)),
        out_shape=jax.ShapeDtypeStruct((N, H), f32),
    )(gene_expr, wgT, bg, coords, wcT, bc)

    idx, w, idxT, wT, rowsum = pl.pallas_call(
        _knn_body,
        grid=(NB,),
        in_specs=[_rowblk((BN, 2)), _full((2, N))],
        out_specs=[_rowblk((BN, KPAD)), _rowblk((BN, KPAD)),
                   pl.BlockSpec((KPAD, BN), lambda b: (0, b)),
                   pl.BlockSpec((KPAD, BN), lambda b: (0, b)),
                   _rowblk((BN, 1))],
        out_shape=[jax.ShapeDtypeStruct((N, KPAD), jnp.int32),
                   jax.ShapeDtypeStruct((N, KPAD), f32),
                   jax.ShapeDtypeStruct((KPAD, N), jnp.int32),
                   jax.ShapeDtypeStruct((KPAD, N), f32),
                   jax.ShapeDtypeStruct((N, 1), f32)],
    )(coords, coordsT)

    colsum = _sc_deg(idx, w)
    deg = 0.5 * (rowsum.reshape(N) + colsum.reshape(N))
    dinv = jnp.power(deg + EPS, -0.5)
    dinv_col = dinv.reshape(N, 1)
    dinv_row = dinv.reshape(1, N)

    z = pl.pallas_call(
        _agg_body,
        grid=(NB,),
        in_specs=[_rowblk((BN, KPAD)), _rowblk((BN, KPAD)), _full((KPAD, N)),
                  _full((KPAD, N)), _full((1, N)), _rowblk((BN, 1)),
                  _full((N, H)), _rowblk((BN, H)), _full((H, H)),
                  _full((1, H)), _full((H, H)), _full((1, H)), _full((H, L)),
                  _full((1, L))],
        out_specs=_rowblk((BN, L)),
        out_shape=jax.ShapeDtypeStruct((N, L), f32),
    )(idx, w, idxT, wT, dinv_row, dinv_col, h, h, wmsgT, bmsg, wselfT, bself,
      wzT, bz)
    return z
